# Initial kernel scaffold; baseline (speedup 1.0000x reference)
import jax
import jax.numpy as jnp
from jax.experimental import pallas as pl


def kernel(boxes, images, patch):
    raise NotImplementedError("write your pallas kernel here")


# speedup vs baseline: 441.5056x; 441.5056x over previous
import functools

import jax
import jax.numpy as jnp
from jax import lax
from jax.experimental import pallas as pl
from jax.experimental.pallas import tpu as pltpu

B = 16
H = 512
W = 512
C = 3
WC = W * C
PS = 240
PSC = PS * C
P = 64
PC = P * C
NB = 20
NBP = 32
MIN_PATCH_AREA = 60.0
SCALE = 0.3


def _rng_consts():
    keys = jax.random.split(jax.random.key(42), B)
    ws, bs, noises = [], [], []
    for i in range(B):
        kw, kb, kn, kbr = jax.random.split(keys[i], 4)
        ws.append(jax.random.normal(kw, (1, 1, 3)) * 0.01 + 0.7)
        bs.append(jax.random.normal(kb, (1, 1, 3)) * 0.01 - 0.3)
        nz = jax.random.uniform(kn, (P, P, 3), minval=-0.1, maxval=0.1)
        br = jax.random.uniform(kbr, (), minval=-0.3, maxval=0.3)
        noises.append(nz + br)
    w = jnp.stack(ws).reshape(B, 3)
    b = jnp.stack(bs).reshape(B, 3)
    noise = jnp.stack(noises).reshape(B, P, PC)
    w_row = jnp.tile(w, (1, PS)).reshape(B, 1, PSC)
    b_row = jnp.tile(b, (1, PS)).reshape(B, 1, PSC)
    return w_row, b_row, noise


def _resize_mats():
    wm = jax.image.resize(jnp.eye(PS, dtype=jnp.float32), (P, PS),
                          method="linear")
    wit = jnp.einsum("jx,pq->xpjq", wm, jnp.eye(3, dtype=jnp.float32))
    return wm, wit.reshape(PSC, PC)


def _stamp_body(img_ref, patch_ref, w_ref, b_ref, noise_ref,
                b0_ref, b1_ref, b2_ref, b3_ref, wm_ref, wit_ref,
                im_ref, y0_ref, x0_ref, val_ref):
    img = img_ref[0]
    mean_img = jnp.sum(img) / (H * W * C)
    p1 = jnp.clip(w_ref[0] * patch_ref[...] + b_ref[0], -1.0, 1.0)
    mean_p = jnp.sum(p1) / (PS * PS * C)
    p2 = jnp.clip(p1 + (mean_img - mean_p), -1.0, 1.0)
    r = jnp.dot(wm_ref[...], p2, preferred_element_type=jnp.float32)
    im = jnp.dot(r, wit_ref[...], preferred_element_type=jnp.float32)
    im_ref[0] = jnp.clip(im + noise_ref[0], -1.0, 1.0)
    a0, a1, a2, a3 = b0_ref[0], b1_ref[0], b2_ref[0], b3_ref[0]
    y0 = jnp.minimum(a0, a2) * float(H)
    y1 = jnp.maximum(a0, a2) * float(H)
    x0 = jnp.minimum(a1, a3) * float(W)
    x1 = jnp.maximum(a1, a3) * float(W)
    h = y1 - y0
    w = x1 - x0
    ps = jnp.floor(jnp.sqrt(h * w * SCALE))
    ymin = jnp.maximum(y0 + h * 0.5 - ps * 0.5, 0.0)
    xmin = jnp.maximum(x0 + w * 0.5 - ps * 0.5, 0.0)
    ymin = jnp.where(ymin + ps > float(H), float(H) - ps, ymin)
    xmin = jnp.where(xmin + ps > float(W), float(W) - ps, xmin)
    y0_ref[0] = jnp.clip(ymin.astype(jnp.int32), 0, H - P)
    x0_ref[0] = jnp.clip(xmin.astype(jnp.int32), 0, W - P)
    val_ref[0] = (ps * ps > MIN_PATCH_AREA).astype(jnp.int32)


def _stamps(images2, patch2, boxes):
    w_row, b_row, noise = _rng_consts()
    wm, wit = _resize_mats()
    bc = jnp.pad(boxes, ((0, 0), (0, NBP - NB), (0, 0)))
    bcs = [bc[:, :, k].reshape(B, 1, NBP) for k in range(4)]
    one = lambda i: (i, 0, 0)
    fixed2 = pl.BlockSpec((PS, PSC), lambda i: (0, 0))
    return pl.pallas_call(
        _stamp_body,
        grid=(B,),
        in_specs=[
            pl.BlockSpec((1, H, WC), one),
            pl.BlockSpec((PS, PSC), lambda i: (0, 0)),
            pl.BlockSpec((1, 1, PSC), one),
            pl.BlockSpec((1, 1, PSC), one),
            pl.BlockSpec((1, P, PC), one),
            pl.BlockSpec((1, 1, NBP), one),
            pl.BlockSpec((1, 1, NBP), one),
            pl.BlockSpec((1, 1, NBP), one),
            pl.BlockSpec((1, 1, NBP), one),
            pl.BlockSpec((P, PS), lambda i: (0, 0)),
            pl.BlockSpec((PSC, PC), lambda i: (0, 0)),
        ],
        out_specs=[
            pl.BlockSpec((1, P, PC), one),
            pl.BlockSpec((1, 1, NBP), one),
            pl.BlockSpec((1, 1, NBP), one),
            pl.BlockSpec((1, 1, NBP), one),
        ],
        out_shape=[
            jax.ShapeDtypeStruct((B, P, PC), jnp.float32),
            jax.ShapeDtypeStruct((B, 1, NBP), jnp.int32),
            jax.ShapeDtypeStruct((B, 1, NBP), jnp.int32),
            jax.ShapeDtypeStruct((B, 1, NBP), jnp.int32),
        ],
    )(images2, patch2, w_row, b_row, noise, *bcs, wm, wit)


WH = 80
WW = 384


def _scatter_body(y0_s, x0_s, val_s, img_ref, im_ref, out_ref, mask_ref):
    i = pl.program_id(0)
    out_ref[0] = img_ref[0]
    mask_ref[0] = jnp.zeros((H, WC), jnp.float32)
    spad = jnp.pad(im_ref[0], ((0, WH - P), (0, WW - PC)))
    rows = lax.broadcasted_iota(jnp.int32, (WH, WW), 0)
    cols = lax.broadcasted_iota(jnp.int32, (WH, WW), 1)

    def box(j, carry):
        y0 = y0_s[i, 0, j]
        xc = x0_s[i, 0, j] * 3

        @pl.when(val_s[i, 0, j] == 1)
        def _():
            ws = pl.multiple_of(jnp.minimum((y0 // 8) * 8, H - WH), 8)
            wsx = pl.multiple_of(jnp.minimum((xc // 128) * 128, WC - WW), 128)
            dy = y0 - ws
            dx = xc - wsx
            rolled = pltpu.roll(pltpu.roll(spad, dy, 0), dx, 1)
            hit = (rows >= dy) & (rows < dy + P) & (cols >= dx) & (cols < dx + PC)
            cur = out_ref[0, pl.ds(ws, WH), pl.ds(wsx, WW)]
            out_ref[0, pl.ds(ws, WH), pl.ds(wsx, WW)] = jnp.where(hit, rolled, cur)
            orig = img_ref[0, pl.ds(ws, WH), pl.ds(wsx, WW)]
            mcur = mask_ref[0, pl.ds(ws, WH), pl.ds(wsx, WW)]
            mask_ref[0, pl.ds(ws, WH), pl.ds(wsx, WW)] = jnp.where(
                hit, orig - rolled, mcur)

        return carry

    lax.fori_loop(0, NB, box, 0)


def _scatter(images2, im, y0i, x0i, vali):
    one = lambda i: (i, 0, 0)
    smem = pl.BlockSpec(memory_space=pltpu.SMEM)
    return pl.pallas_call(
        _scatter_body,
        grid=(B,),
        in_specs=[
            smem, smem, smem,
            pl.BlockSpec((1, H, WC), one),
            pl.BlockSpec((1, P, PC), one),
        ],
        out_specs=[
            pl.BlockSpec((1, H, WC), one),
            pl.BlockSpec((1, H, WC), one),
        ],
        out_shape=[
            jax.ShapeDtypeStruct((B, H, WC), jnp.float32),
            jax.ShapeDtypeStruct((B, H, WC), jnp.float32),
        ],
    )(y0i, x0i, vali, images2, im)


def kernel(boxes, images, patch):
    images2 = images.reshape(B, H, WC)
    patch2 = patch.reshape(PS, PSC)
    im, y0i, x0i, vali = _stamps(images2, patch2, boxes)
    out, mask = _scatter(images2, im, y0i, x0i, vali)
    return out.reshape(B, H, W, C), mask.reshape(B, H, W, C)
